# Initial kernel scaffold; baseline (speedup 1.0000x reference)
import jax
import jax.numpy as jnp
from jax.experimental import pallas as pl


def kernel(x, edge_index, edge_attr, u, batch, W1, b1, W2, b2):
    raise NotImplementedError("write your pallas kernel here")


# speedup vs baseline: 3.5860x; 3.5860x over previous
import functools

import jax
import jax.numpy as jnp
from jax import lax
from jax.experimental import pallas as pl
from jax.experimental.pallas import tpu as pltpu
from jax.experimental.pallas import tpu_sc as plsc

N_NODES = 10000
N_EDGES = 320000
D = 128

NC, NS = 2, 16
NW = NC * NS
EPW = N_EDGES // NW
BLK = 80
NBLK = EPW // BLK
CH = 80
NCH = BLK // CH
N_PAD = 10240
RPT = N_PAD // NS


def _sc_segment_sum(edge_attr, idx):
    mesh = plsc.VectorSubcoreMesh(core_axis_name="c", subcore_axis_name="s")

    @functools.partial(
        pl.kernel,
        out_type=jax.ShapeDtypeStruct((NC, N_PAD, D), jnp.float32),
        mesh=mesh,
        scratch_types=[
            pltpu.VMEM_SHARED((N_PAD, D), jnp.float32),
            pltpu.VMEM((BLK, D), jnp.float32),
            pltpu.VMEM((NCH, CH), jnp.int32),
        ],
    )
    def sc_kernel(e_hbm, i_hbm, o_hbm, acc, ebuf, ibuf):
        cid = lax.axis_index("c")
        sid = lax.axis_index("s")
        wid = sid * NC + cid

        @pl.loop(0, BLK)
        def _(r):
            @pl.loop(0, D, step=16)
            def _(j):
                ebuf[r, pl.ds(j, 16)] = jnp.zeros((16,), jnp.float32)

        @pl.loop(0, RPT, step=BLK)
        def _(r0):
            pltpu.sync_copy(ebuf, acc.at[pl.ds(sid * RPT + r0, BLK)])

        plsc.subcore_barrier()

        @pl.loop(0, NBLK)
        def _(b):
            base = wid * EPW + b * BLK
            pltpu.sync_copy(e_hbm.at[pl.ds(base, BLK)], ebuf)
            pltpu.sync_copy(i_hbm.at[wid, b], ibuf)
            for j in range(NCH):
                pltpu.sync_copy(
                    ebuf.at[pl.ds(j * CH, CH)], acc.at[ibuf.at[j]], add=True
                )

        plsc.subcore_barrier()

        pltpu.sync_copy(
            acc.at[pl.ds(sid * RPT, RPT)], o_hbm.at[cid, pl.ds(sid * RPT, RPT)]
        )

    return sc_kernel(edge_attr, idx)


def _tc_mlp(x, partials, W1, b1, W2, b2):
    BR = 1000

    def body(x_ref, p_ref, w1_ref, b1_ref, w2_ref, b2_ref, o_ref):
        aggr = p_ref[0] + p_ref[1]
        h = jnp.dot(x_ref[...], w1_ref[:D], preferred_element_type=jnp.float32)
        h += jnp.dot(aggr, w1_ref[D:], preferred_element_type=jnp.float32)
        h = jnp.maximum(h + b1_ref[...], 0.0)
        o_ref[...] = (
            jnp.dot(h, w2_ref[...], preferred_element_type=jnp.float32) + b2_ref[...]
        )

    return pl.pallas_call(
        body,
        grid=(N_NODES // BR,),
        in_specs=[
            pl.BlockSpec((BR, D), lambda i: (i, 0)),
            pl.BlockSpec((NC, BR, D), lambda i: (0, i, 0)),
            pl.BlockSpec((2 * D, D), lambda i: (0, 0)),
            pl.BlockSpec((1, D), lambda i: (0, 0)),
            pl.BlockSpec((D, D), lambda i: (0, 0)),
            pl.BlockSpec((1, D), lambda i: (0, 0)),
        ],
        out_specs=pl.BlockSpec((BR, D), lambda i: (i, 0)),
        out_shape=jax.ShapeDtypeStruct((N_NODES, D), jnp.float32),
    )(x, partials, W1, b1, W2, b2)


def kernel(x, edge_index, edge_attr, u, batch, W1, b1, W2, b2):
    row = edge_index[0].astype(jnp.int32).reshape(NW, NBLK, NCH, CH)
    partials = _sc_segment_sum(edge_attr, row)
    return _tc_mlp(
        x, partials, W1, b1.reshape(1, D), W2, b2.reshape(1, D)
    )


# speedup vs baseline: 6.8160x; 1.9008x over previous
import functools

import jax
import jax.numpy as jnp
from jax import lax
from jax.experimental import pallas as pl
from jax.experimental.pallas import tpu as pltpu
from jax.experimental.pallas import tpu_sc as plsc

N_NODES = 10000
N_EDGES = 320000
D = 128

NC, NS = 2, 16
NW = NC * NS
EPW = N_EDGES // NW
BLK = 80
NBLK = EPW // BLK
CH = 80
NCH = BLK // CH
N_PAD = 10240
RPT = N_PAD // NS


def _sc_segment_sum(edge_attr, idx):
    mesh = plsc.VectorSubcoreMesh(core_axis_name="c", subcore_axis_name="s")

    @functools.partial(
        pl.kernel,
        out_type=jax.ShapeDtypeStruct((NC, N_PAD, D), jnp.float32),
        mesh=mesh,
        scratch_types=[
            pltpu.VMEM_SHARED((N_PAD, D), jnp.float32),
            pltpu.VMEM((BLK, D), jnp.float32),
            pltpu.VMEM((BLK, D), jnp.float32),
            pltpu.VMEM((NCH, CH), jnp.int32),
            pltpu.VMEM((NCH, CH), jnp.int32),
            pltpu.SemaphoreType.DMA,
            pltpu.SemaphoreType.DMA,
        ],
    )
    def sc_kernel(e_hbm, i_hbm, o_hbm, acc, ebuf0, ebuf1, ibuf0, ibuf1, sem0, sem1):
        cid = lax.axis_index("c")
        sid = lax.axis_index("s")
        wid = sid * NC + cid

        @pl.loop(0, BLK)
        def _(r):
            @pl.loop(0, D, step=16)
            def _(j):
                ebuf0[r, pl.ds(j, 16)] = jnp.zeros((16,), jnp.float32)

        @pl.loop(0, RPT, step=BLK)
        def _(r0):
            pltpu.sync_copy(ebuf0, acc.at[pl.ds(sid * RPT + r0, BLK)])

        plsc.subcore_barrier()

        def start(b, ebuf, ibuf, sem):
            base = wid * EPW + b * BLK
            pltpu.async_copy(e_hbm.at[pl.ds(base, BLK)], ebuf, sem)
            pltpu.async_copy(i_hbm.at[wid, b], ibuf, sem)

        def wait(ebuf, ibuf, sem):
            pltpu.make_async_copy(e_hbm.at[pl.ds(0, BLK)], ebuf, sem).wait()
            pltpu.make_async_copy(i_hbm.at[0, 0], ibuf, sem).wait()

        def stream(ebuf, ibuf):
            for j in range(NCH):
                pltpu.sync_copy(
                    ebuf.at[pl.ds(j * CH, CH)], acc.at[ibuf.at[j]], add=True
                )

        start(0, ebuf0, ibuf0, sem0)
        start(1, ebuf1, ibuf1, sem1)

        @pl.loop(0, NBLK - 1, step=2)
        def _(g):
            wait(ebuf0, ibuf0, sem0)
            stream(ebuf0, ibuf0)
            start(g + 2, ebuf0, ibuf0, sem0)
            wait(ebuf1, ibuf1, sem1)
            stream(ebuf1, ibuf1)

            @pl.when(g + 3 < NBLK)
            def _():
                start(g + 3, ebuf1, ibuf1, sem1)

        wait(ebuf0, ibuf0, sem0)
        stream(ebuf0, ibuf0)

        plsc.subcore_barrier()

        pltpu.sync_copy(
            acc.at[pl.ds(sid * RPT, RPT)], o_hbm.at[cid, pl.ds(sid * RPT, RPT)]
        )

    return sc_kernel(edge_attr, idx)


def _tc_mlp(x, partials, W1, b1, W2, b2):
    BR = 1000

    def body(x_ref, p_ref, w1_ref, b1_ref, w2_ref, b2_ref, o_ref):
        aggr = p_ref[0] + p_ref[1]
        h = jnp.dot(x_ref[...], w1_ref[:D], preferred_element_type=jnp.float32)
        h += jnp.dot(aggr, w1_ref[D:], preferred_element_type=jnp.float32)
        h = jnp.maximum(h + b1_ref[...], 0.0)
        o_ref[...] = (
            jnp.dot(h, w2_ref[...], preferred_element_type=jnp.float32) + b2_ref[...]
        )

    return pl.pallas_call(
        body,
        grid=(N_NODES // BR,),
        in_specs=[
            pl.BlockSpec((BR, D), lambda i: (i, 0)),
            pl.BlockSpec((NC, BR, D), lambda i: (0, i, 0)),
            pl.BlockSpec((2 * D, D), lambda i: (0, 0)),
            pl.BlockSpec((1, D), lambda i: (0, 0)),
            pl.BlockSpec((D, D), lambda i: (0, 0)),
            pl.BlockSpec((1, D), lambda i: (0, 0)),
        ],
        out_specs=pl.BlockSpec((BR, D), lambda i: (i, 0)),
        out_shape=jax.ShapeDtypeStruct((N_NODES, D), jnp.float32),
    )(x, partials, W1, b1, W2, b2)


def kernel(x, edge_index, edge_attr, u, batch, W1, b1, W2, b2):
    row = edge_index[0].astype(jnp.int32).reshape(NW, NBLK, NCH, CH)
    partials = _sc_segment_sum(edge_attr, row)
    return _tc_mlp(
        x, partials, W1, b1.reshape(1, D), W2, b2.reshape(1, D)
    )


# speedup vs baseline: 7.8982x; 1.1588x over previous
import functools

import jax
import jax.numpy as jnp
from jax import lax
from jax.experimental import pallas as pl
from jax.experimental.pallas import tpu as pltpu
from jax.experimental.pallas import tpu_sc as plsc

N_NODES = 10000
N_EDGES = 320000
D = 128

NC, NS = 2, 16
NW = NC * NS
EPW = N_EDGES // NW
BLK = 80
NBLK = EPW // BLK
CH = 80
NCH = BLK // CH
N_PAD = 10240
RPT = N_PAD // NS


def _sc_segment_sum(edge_attr, idx):
    mesh = plsc.VectorSubcoreMesh(core_axis_name="c", subcore_axis_name="s")

    @functools.partial(
        pl.kernel,
        out_type=jax.ShapeDtypeStruct((NC, N_PAD, D), jnp.float32),
        mesh=mesh,
        scratch_types=[
            pltpu.VMEM_SHARED((N_PAD, D), jnp.float32),
            pltpu.VMEM((BLK, D), jnp.float32),
            pltpu.VMEM((BLK, D), jnp.float32),
            pltpu.VMEM((BLK, D), jnp.float32),
            pltpu.VMEM((BLK, D), jnp.float32),
            pltpu.VMEM((NCH, CH), jnp.int32),
            pltpu.VMEM((NCH, CH), jnp.int32),
            pltpu.VMEM((NCH, CH), jnp.int32),
            pltpu.VMEM((NCH, CH), jnp.int32),
            pltpu.SemaphoreType.DMA,
            pltpu.SemaphoreType.DMA,
            pltpu.SemaphoreType.DMA,
            pltpu.SemaphoreType.DMA,
        ],
    )
    def sc_kernel(
        e_hbm, i_hbm, o_hbm, acc,
        ebuf0, ebuf1, ebuf2, ebuf3,
        ibuf0, ibuf1, ibuf2, ibuf3,
        sem0, sem1, sem2, sem3,
    ):
        cid = lax.axis_index("c")
        sid = lax.axis_index("s")
        wid = sid * NC + cid

        @pl.loop(0, BLK)
        def _(r):
            @pl.loop(0, D, step=16)
            def _(j):
                ebuf0[r, pl.ds(j, 16)] = jnp.zeros((16,), jnp.float32)

        @pl.loop(0, RPT, step=BLK)
        def _(r0):
            pltpu.sync_copy(ebuf0, acc.at[pl.ds(sid * RPT + r0, BLK)])

        plsc.subcore_barrier()

        def start(b, ebuf, ibuf, sem):
            base = wid * EPW + b * BLK
            pltpu.async_copy(e_hbm.at[pl.ds(base, BLK)], ebuf, sem)
            pltpu.async_copy(i_hbm.at[wid, b], ibuf, sem)

        def wait(ebuf, ibuf, sem):
            pltpu.make_async_copy(e_hbm.at[pl.ds(0, BLK)], ebuf, sem).wait()
            pltpu.make_async_copy(i_hbm.at[0, 0], ibuf, sem).wait()

        def stream(ebuf, ibuf):
            for j in range(NCH):
                pltpu.sync_copy(
                    ebuf.at[pl.ds(j * CH, CH)], acc.at[ibuf.at[j]], add=True
                )

        bufs = (
            (ebuf0, ibuf0, sem0),
            (ebuf1, ibuf1, sem1),
            (ebuf2, ibuf2, sem2),
            (ebuf3, ibuf3, sem3),
        )
        for k in range(4):
            start(k, *bufs[k])

        @pl.loop(0, NBLK - 1, step=4)
        def _(g):
            for b in range(4):
                wait(*bufs[b])
                stream(bufs[b][0], bufs[b][1])
                nxt = g + 4 + b

                @pl.when(nxt < NBLK)
                def _():
                    start(nxt, *bufs[b])

        wait(*bufs[0])
        stream(bufs[0][0], bufs[0][1])

        plsc.subcore_barrier()

        pltpu.sync_copy(
            acc.at[pl.ds(sid * RPT, RPT)], o_hbm.at[cid, pl.ds(sid * RPT, RPT)]
        )

    return sc_kernel(edge_attr, idx)


def _tc_mlp(x, partials, W1, b1, W2, b2):
    BR = 1000

    def body(x_ref, p_ref, w1_ref, b1_ref, w2_ref, b2_ref, o_ref):
        aggr = p_ref[0] + p_ref[1]
        h = jnp.dot(x_ref[...], w1_ref[:D], preferred_element_type=jnp.float32)
        h += jnp.dot(aggr, w1_ref[D:], preferred_element_type=jnp.float32)
        h = jnp.maximum(h + b1_ref[...], 0.0)
        o_ref[...] = (
            jnp.dot(h, w2_ref[...], preferred_element_type=jnp.float32) + b2_ref[...]
        )

    return pl.pallas_call(
        body,
        grid=(N_NODES // BR,),
        in_specs=[
            pl.BlockSpec((BR, D), lambda i: (i, 0)),
            pl.BlockSpec((NC, BR, D), lambda i: (0, i, 0)),
            pl.BlockSpec((2 * D, D), lambda i: (0, 0)),
            pl.BlockSpec((1, D), lambda i: (0, 0)),
            pl.BlockSpec((D, D), lambda i: (0, 0)),
            pl.BlockSpec((1, D), lambda i: (0, 0)),
        ],
        out_specs=pl.BlockSpec((BR, D), lambda i: (i, 0)),
        out_shape=jax.ShapeDtypeStruct((N_NODES, D), jnp.float32),
    )(x, partials, W1, b1, W2, b2)


def kernel(x, edge_index, edge_attr, u, batch, W1, b1, W2, b2):
    row = edge_index[0].astype(jnp.int32).reshape(NW, NBLK, NCH, CH)
    partials = _sc_segment_sum(edge_attr, row)
    return _tc_mlp(
        x, partials, W1, b1.reshape(1, D), W2, b2.reshape(1, D)
    )


# speedup vs baseline: 8.2185x; 1.0406x over previous
import functools

import jax
import jax.numpy as jnp
from jax import lax
from jax.experimental import pallas as pl
from jax.experimental.pallas import tpu as pltpu
from jax.experimental.pallas import tpu_sc as plsc

N_NODES = 10000
N_EDGES = 320000
D = 128

NC, NS = 2, 16
NW = NC * NS
EPW = N_EDGES // NW
BLK = 80
NBLK = EPW // BLK
CH = 80
NCH = BLK // CH
N_PAD = 10240
RPT = N_PAD // NS


def _sc_segment_sum(edge_attr, idx):
    mesh = plsc.VectorSubcoreMesh(core_axis_name="c", subcore_axis_name="s")

    @functools.partial(
        pl.kernel,
        out_type=jax.ShapeDtypeStruct((NC, N_PAD, D), jnp.float32),
        mesh=mesh,
        scratch_types=[
            pltpu.VMEM_SHARED((N_PAD, D), jnp.float32),
            pltpu.VMEM((BLK, D), jnp.float32),
            pltpu.VMEM((BLK, D), jnp.float32),
            pltpu.VMEM((BLK, D), jnp.float32),
            pltpu.VMEM((BLK, D), jnp.float32),
            pltpu.VMEM((BLK,), jnp.int32),
            pltpu.VMEM((BLK,), jnp.int32),
            pltpu.VMEM((BLK,), jnp.int32),
            pltpu.VMEM((BLK,), jnp.int32),
            pltpu.SemaphoreType.DMA,
            pltpu.SemaphoreType.DMA,
            pltpu.SemaphoreType.DMA,
            pltpu.SemaphoreType.DMA,
        ],
    )
    def sc_kernel(
        e_hbm, i_hbm, o_hbm, acc,
        ebuf0, ebuf1, ebuf2, ebuf3,
        ibuf0, ibuf1, ibuf2, ibuf3,
        sem0, sem1, sem2, sem3,
    ):
        cid = lax.axis_index("c")
        sid = lax.axis_index("s")
        wid = sid * NC + cid

        @pl.loop(0, BLK)
        def _(r):
            @pl.loop(0, D, step=16)
            def _(j):
                ebuf0[r, pl.ds(j, 16)] = jnp.zeros((16,), jnp.float32)

        @pl.loop(0, RPT, step=BLK)
        def _(r0):
            pltpu.sync_copy(ebuf0, acc.at[pl.ds(sid * RPT + r0, BLK)])

        plsc.subcore_barrier()

        def start(b, ebuf, ibuf, sem):
            base = wid * EPW + b * BLK
            pltpu.async_copy(e_hbm.at[pl.ds(base, BLK)], ebuf, sem)
            pltpu.async_copy(i_hbm.at[pl.ds(base, BLK)], ibuf, sem)

        def wait(ebuf, ibuf, sem):
            pltpu.make_async_copy(e_hbm.at[pl.ds(0, BLK)], ebuf, sem).wait()
            pltpu.make_async_copy(i_hbm.at[pl.ds(0, BLK)], ibuf, sem).wait()

        def stream(ebuf, ibuf):
            pltpu.sync_copy(ebuf, acc.at[ibuf], add=True)

        bufs = (
            (ebuf0, ibuf0, sem0),
            (ebuf1, ibuf1, sem1),
            (ebuf2, ibuf2, sem2),
            (ebuf3, ibuf3, sem3),
        )
        for k in range(4):
            start(k, *bufs[k])

        @pl.loop(0, NBLK - 1, step=4)
        def _(g):
            for b in range(4):
                wait(*bufs[b])
                stream(bufs[b][0], bufs[b][1])
                nxt = g + 4 + b

                @pl.when(nxt < NBLK)
                def _():
                    start(nxt, *bufs[b])

        wait(*bufs[0])
        stream(bufs[0][0], bufs[0][1])

        plsc.subcore_barrier()

        pltpu.sync_copy(
            acc.at[pl.ds(sid * RPT, RPT)], o_hbm.at[cid, pl.ds(sid * RPT, RPT)]
        )

    return sc_kernel(edge_attr, idx)


BR = 1000


def _tc_pre(x, W1, b1):
    def body(x_ref, w1_ref, b1_ref, t_ref):
        t_ref[...] = (
            jnp.dot(x_ref[...], w1_ref[...], preferred_element_type=jnp.float32)
            + b1_ref[...]
        )

    return pl.pallas_call(
        body,
        grid=(N_NODES // BR,),
        in_specs=[
            pl.BlockSpec((BR, D), lambda i: (i, 0)),
            pl.BlockSpec((D, D), lambda i: (0, 0)),
            pl.BlockSpec((1, D), lambda i: (0, 0)),
        ],
        out_specs=pl.BlockSpec((BR, D), lambda i: (i, 0)),
        out_shape=jax.ShapeDtypeStruct((N_NODES, D), jnp.float32),
    )(x, W1, b1)


def _tc_post(t, partials, W1b, W2, b2):
    def body(t_ref, p_ref, w1b_ref, w2_ref, b2_ref, o_ref):
        aggr = p_ref[0] + p_ref[1]
        h = t_ref[...] + jnp.dot(
            aggr, w1b_ref[...], preferred_element_type=jnp.float32
        )
        h = jnp.maximum(h, 0.0)
        o_ref[...] = (
            jnp.dot(h, w2_ref[...], preferred_element_type=jnp.float32) + b2_ref[...]
        )

    return pl.pallas_call(
        body,
        grid=(N_NODES // BR,),
        in_specs=[
            pl.BlockSpec((BR, D), lambda i: (i, 0)),
            pl.BlockSpec((NC, BR, D), lambda i: (0, i, 0)),
            pl.BlockSpec((D, D), lambda i: (0, 0)),
            pl.BlockSpec((D, D), lambda i: (0, 0)),
            pl.BlockSpec((1, D), lambda i: (0, 0)),
        ],
        out_specs=pl.BlockSpec((BR, D), lambda i: (i, 0)),
        out_shape=jax.ShapeDtypeStruct((N_NODES, D), jnp.float32),
    )(t, partials, W1b, W2, b2)


def kernel(x, edge_index, edge_attr, u, batch, W1, b1, W2, b2):
    idx = edge_index[0].astype(jnp.int32)
    partials = _sc_segment_sum(edge_attr, idx)
    t = _tc_pre(x, W1[:D], b1.reshape(1, D))
    return _tc_post(t, partials, W1[D:], W2, b2.reshape(1, D))


# speedup vs baseline: 8.9922x; 1.0941x over previous
import functools

import jax
import jax.numpy as jnp
from jax import lax
from jax.experimental import pallas as pl
from jax.experimental.pallas import tpu as pltpu
from jax.experimental.pallas import tpu_sc as plsc

N_NODES = 10000
N_EDGES = 320000
D = 128

NC, NS = 2, 16
NW = NC * NS
BLK = 128
NBLOCKS = N_EDGES // BLK
SLOTS = 79
NBUF = 3
N_PAD = 10112
RPT = N_PAD // NS


def _sc_segment_sum(edge_attr, edge_index):
    mesh = plsc.VectorSubcoreMesh(core_axis_name="c", subcore_axis_name="s")

    @functools.partial(
        pl.kernel,
        out_type=jax.ShapeDtypeStruct((NC, N_PAD, D), jnp.float32),
        mesh=mesh,
        scratch_types=[
            pltpu.VMEM_SHARED((N_PAD, D), jnp.float32),
            pltpu.VMEM((BLK, D), jnp.float32),
            pltpu.VMEM((BLK, D), jnp.float32),
            pltpu.VMEM((BLK, D), jnp.float32),
            pltpu.VMEM((2, BLK), jnp.int32),
            pltpu.VMEM((2, BLK), jnp.int32),
            pltpu.VMEM((2, BLK), jnp.int32),
            pltpu.SemaphoreType.DMA,
            pltpu.SemaphoreType.DMA,
            pltpu.SemaphoreType.DMA,
        ],
    )
    def sc_kernel(
        e_hbm, i_hbm, o_hbm, acc,
        ebuf0, ebuf1, ebuf2,
        ibuf0, ibuf1, ibuf2,
        sem0, sem1, sem2,
    ):
        cid = lax.axis_index("c")
        sid = lax.axis_index("s")
        wid = sid * NC + cid
        base_blk = wid * SLOTS

        @pl.loop(0, BLK)
        def _(r):
            @pl.loop(0, D, step=16)
            def _(j):
                ebuf0[r, pl.ds(j, 16)] = jnp.zeros((16,), jnp.float32)

        @pl.loop(0, RPT - BLK + 1, step=BLK)
        def _(r0):
            pltpu.sync_copy(ebuf0, acc.at[pl.ds(sid * RPT + r0, BLK)])

        pltpu.sync_copy(
            ebuf0.at[pl.ds(0, RPT - 4 * BLK)],
            acc.at[pl.ds(sid * RPT + 4 * BLK, RPT - 4 * BLK)],
        )

        plsc.subcore_barrier()

        def start(slot, ebuf, ibuf, sem):
            g = base_blk + slot

            @pl.when(jnp.logical_and(slot < SLOTS, g < NBLOCKS))
            def _():
                pltpu.async_copy(e_hbm.at[pl.ds(g * BLK, BLK)], ebuf, sem)
                pltpu.async_copy(i_hbm.at[:, pl.ds(g * BLK, BLK)], ibuf, sem)

        def finish(slot, ebuf, ibuf, sem):
            g = base_blk + slot

            @pl.when(g < NBLOCKS)
            def _():
                pltpu.make_async_copy(e_hbm.at[pl.ds(0, BLK)], ebuf, sem).wait()
                pltpu.make_async_copy(
                    i_hbm.at[:, pl.ds(0, BLK)], ibuf, sem
                ).wait()
                pltpu.sync_copy(ebuf, acc.at[ibuf.at[0]], add=True)

        bufs = (
            (ebuf0, ibuf0, sem0),
            (ebuf1, ibuf1, sem1),
            (ebuf2, ibuf2, sem2),
        )
        for k in range(NBUF):
            start(k, *bufs[k])

        @pl.loop(0, SLOTS - 1, step=NBUF)
        def _(s0):
            for b in range(NBUF):
                slot = s0 + b
                finish(slot, *bufs[b])
                start(slot + NBUF, *bufs[b])

        finish(SLOTS - 1, *bufs[(SLOTS - 1) % NBUF])

        plsc.subcore_barrier()

        pltpu.sync_copy(
            acc.at[pl.ds(sid * RPT, RPT)], o_hbm.at[cid, pl.ds(sid * RPT, RPT)]
        )

    return sc_kernel(edge_attr, edge_index)


BR = 1000


def _tc_pre(x, W1, b1):
    def body(x_ref, w1_ref, b1_ref, t_ref):
        t_ref[...] = (
            jnp.dot(x_ref[...], w1_ref[...], preferred_element_type=jnp.float32)
            + b1_ref[...]
        )

    return pl.pallas_call(
        body,
        grid=(N_NODES // BR,),
        in_specs=[
            pl.BlockSpec((BR, D), lambda i: (i, 0)),
            pl.BlockSpec((D, D), lambda i: (0, 0)),
            pl.BlockSpec((1, D), lambda i: (0, 0)),
        ],
        out_specs=pl.BlockSpec((BR, D), lambda i: (i, 0)),
        out_shape=jax.ShapeDtypeStruct((N_NODES, D), jnp.float32),
    )(x, W1, b1)


def _tc_post(t, partials, W1b, W2, b2):
    def body(t_ref, p_ref, w1b_ref, w2_ref, b2_ref, o_ref):
        aggr = p_ref[0] + p_ref[1]
        h = t_ref[...] + jnp.dot(
            aggr, w1b_ref[...], preferred_element_type=jnp.float32
        )
        h = jnp.maximum(h, 0.0)
        o_ref[...] = (
            jnp.dot(h, w2_ref[...], preferred_element_type=jnp.float32) + b2_ref[...]
        )

    return pl.pallas_call(
        body,
        grid=(N_NODES // BR,),
        in_specs=[
            pl.BlockSpec((BR, D), lambda i: (i, 0)),
            pl.BlockSpec((NC, BR, D), lambda i: (0, i, 0)),
            pl.BlockSpec((D, D), lambda i: (0, 0)),
            pl.BlockSpec((D, D), lambda i: (0, 0)),
            pl.BlockSpec((1, D), lambda i: (0, 0)),
        ],
        out_specs=pl.BlockSpec((BR, D), lambda i: (i, 0)),
        out_shape=jax.ShapeDtypeStruct((N_NODES, D), jnp.float32),
    )(t, partials, W1b, W2, b2)


def kernel(x, edge_index, edge_attr, u, batch, W1, b1, W2, b2):
    partials = _sc_segment_sum(edge_attr, edge_index.astype(jnp.int32))
    t = _tc_pre(x, W1[:D], b1.reshape(1, D))
    return _tc_post(t, partials, W1[D:], W2, b2.reshape(1, D))


# speedup vs baseline: 8.9970x; 1.0005x over previous
import functools

import jax
import jax.numpy as jnp
from jax import lax
from jax.experimental import pallas as pl
from jax.experimental.pallas import tpu as pltpu
from jax.experimental.pallas import tpu_sc as plsc

N_NODES = 10000
N_EDGES = 320000
D = 128

NC, NS = 2, 16
NW = NC * NS
BLK = 128
NBLOCKS = N_EDGES // BLK
SLOTS = 79
NBUF = 3
N_PAD = 10112
RPT = N_PAD // NS


def _sc_segment_sum(edge_attr, edge_index):
    mesh = plsc.VectorSubcoreMesh(core_axis_name="c", subcore_axis_name="s")

    @functools.partial(
        pl.kernel,
        out_type=jax.ShapeDtypeStruct((NC, N_PAD, D), jnp.float32),
        mesh=mesh,
        scratch_types=[
            pltpu.VMEM_SHARED((N_PAD, D), jnp.float32),
            pltpu.VMEM((BLK, D), jnp.float32),
            pltpu.VMEM((BLK, D), jnp.float32),
            pltpu.VMEM((BLK, D), jnp.float32),
            pltpu.VMEM((1, BLK), jnp.int32),
            pltpu.VMEM((1, BLK), jnp.int32),
            pltpu.VMEM((1, BLK), jnp.int32),
            pltpu.SemaphoreType.DMA,
            pltpu.SemaphoreType.DMA,
            pltpu.SemaphoreType.DMA,
        ],
    )
    def sc_kernel(
        e_hbm, i_hbm, o_hbm, acc,
        ebuf0, ebuf1, ebuf2,
        ibuf0, ibuf1, ibuf2,
        sem0, sem1, sem2,
    ):
        cid = lax.axis_index("c")
        sid = lax.axis_index("s")
        wid = sid * NC + cid
        base_blk = wid * SLOTS

        @pl.loop(0, BLK)
        def _(r):
            @pl.loop(0, D, step=16)
            def _(j):
                ebuf0[r, pl.ds(j, 16)] = jnp.zeros((16,), jnp.float32)

        @pl.loop(0, RPT - BLK + 1, step=BLK)
        def _(r0):
            pltpu.sync_copy(ebuf0, acc.at[pl.ds(sid * RPT + r0, BLK)])

        pltpu.sync_copy(
            ebuf0.at[pl.ds(0, RPT - 4 * BLK)],
            acc.at[pl.ds(sid * RPT + 4 * BLK, RPT - 4 * BLK)],
        )

        plsc.subcore_barrier()

        def start(slot, ebuf, ibuf, sem):
            g = base_blk + slot

            @pl.when(jnp.logical_and(slot < SLOTS, g < NBLOCKS))
            def _():
                pltpu.async_copy(e_hbm.at[pl.ds(g * BLK, BLK)], ebuf, sem)
                pltpu.async_copy(
                    i_hbm.at[pl.ds(0, 1), pl.ds(g * BLK, BLK)], ibuf, sem
                )

        def finish(slot, ebuf, ibuf, sem):
            g = base_blk + slot

            @pl.when(g < NBLOCKS)
            def _():
                pltpu.make_async_copy(e_hbm.at[pl.ds(0, BLK)], ebuf, sem).wait()
                pltpu.make_async_copy(
                    i_hbm.at[pl.ds(0, 1), pl.ds(0, BLK)], ibuf, sem
                ).wait()
                pltpu.sync_copy(ebuf, acc.at[ibuf.at[0]], add=True)

        bufs = (
            (ebuf0, ibuf0, sem0),
            (ebuf1, ibuf1, sem1),
            (ebuf2, ibuf2, sem2),
        )
        for k in range(NBUF):
            start(k, *bufs[k])

        @pl.loop(0, SLOTS - 1, step=NBUF)
        def _(s0):
            for b in range(NBUF):
                slot = s0 + b
                finish(slot, *bufs[b])
                start(slot + NBUF, *bufs[b])

        finish(SLOTS - 1, *bufs[(SLOTS - 1) % NBUF])

        plsc.subcore_barrier()

        pltpu.sync_copy(
            acc.at[pl.ds(sid * RPT, RPT)], o_hbm.at[cid, pl.ds(sid * RPT, RPT)]
        )

    return sc_kernel(edge_attr, edge_index)


BR = 1000


def _tc_mlp(x, partials, W1a, W1b, W2, b1, b2):
    def body(x_ref, p_ref, w1a_ref, w1b_ref, w2_ref, b1_ref, b2_ref, o_ref):
        aggr = p_ref[0] + p_ref[1]
        h = (
            jnp.dot(x_ref[...], w1a_ref[...], preferred_element_type=jnp.float32)
            + jnp.dot(aggr, w1b_ref[...], preferred_element_type=jnp.float32)
            + b1_ref[...]
        )
        h = jnp.maximum(h, 0.0)
        o_ref[...] = (
            jnp.dot(h, w2_ref[...], preferred_element_type=jnp.float32) + b2_ref[...]
        )

    return pl.pallas_call(
        body,
        grid=(N_NODES // BR,),
        in_specs=[
            pl.BlockSpec((BR, D), lambda i: (i, 0)),
            pl.BlockSpec((NC, BR, D), lambda i: (0, i, 0)),
            pl.BlockSpec((D, D), lambda i: (0, 0)),
            pl.BlockSpec((D, D), lambda i: (0, 0)),
            pl.BlockSpec((D, D), lambda i: (0, 0)),
            pl.BlockSpec((1, D), lambda i: (0, 0)),
            pl.BlockSpec((1, D), lambda i: (0, 0)),
        ],
        out_specs=pl.BlockSpec((BR, D), lambda i: (i, 0)),
        out_shape=jax.ShapeDtypeStruct((N_NODES, D), jnp.float32),
    )(x, partials, W1a, W1b, W2, b1, b2)


def kernel(x, edge_index, edge_attr, u, batch, W1, b1, W2, b2):
    partials = _sc_segment_sum(edge_attr, edge_index.astype(jnp.int32))
    return _tc_mlp(
        x, partials, W1[:D], W1[D:], W2, b1.reshape(1, D), b2.reshape(1, D)
    )


# speedup vs baseline: 9.2222x; 1.0250x over previous
import functools

import jax
import jax.numpy as jnp
from jax import lax
from jax.experimental import pallas as pl
from jax.experimental.pallas import tpu as pltpu
from jax.experimental.pallas import tpu_sc as plsc

N_NODES = 10000
N_EDGES = 320000
D = 128

NC, NS = 2, 16
NW = NC * NS
BLK = 128
NBLOCKS = N_EDGES // BLK
SLOTS = 79
NBUF = 3
N_PAD = 10112
RPT = N_PAD // NS


def _sc_segment_sum(edge_attr, edge_index):
    mesh = plsc.VectorSubcoreMesh(core_axis_name="c", subcore_axis_name="s")

    @functools.partial(
        pl.kernel,
        out_type=jax.ShapeDtypeStruct((NC, N_PAD, D), jnp.float32),
        mesh=mesh,
        scratch_types=[
            pltpu.VMEM_SHARED((N_PAD, D), jnp.float32),
            pltpu.VMEM((BLK, D), jnp.float32),
            pltpu.VMEM((BLK, D), jnp.float32),
            pltpu.VMEM((BLK, D), jnp.float32),
            pltpu.VMEM((1, BLK), jnp.int32),
            pltpu.VMEM((1, BLK), jnp.int32),
            pltpu.VMEM((1, BLK), jnp.int32),
            pltpu.SemaphoreType.DMA,
            pltpu.SemaphoreType.DMA,
            pltpu.SemaphoreType.DMA,
        ],
    )
    def sc_kernel(
        e_hbm, i_hbm, o_hbm, acc,
        ebuf0, ebuf1, ebuf2,
        ibuf0, ibuf1, ibuf2,
        sem0, sem1, sem2,
    ):
        cid = lax.axis_index("c")
        sid = lax.axis_index("s")
        wid = sid * NC + cid
        base_blk = wid * SLOTS

        @pl.loop(0, BLK)
        def _(r):
            @pl.loop(0, D, step=16)
            def _(j):
                ebuf0[r, pl.ds(j, 16)] = jnp.zeros((16,), jnp.float32)

        @pl.loop(0, RPT - BLK + 1, step=BLK)
        def _(r0):
            pltpu.sync_copy(ebuf0, acc.at[pl.ds(sid * RPT + r0, BLK)])

        pltpu.sync_copy(
            ebuf0.at[pl.ds(0, RPT - 4 * BLK)],
            acc.at[pl.ds(sid * RPT + 4 * BLK, RPT - 4 * BLK)],
        )

        plsc.subcore_barrier()

        def start(slot, ebuf, ibuf, sem):
            g = base_blk + slot

            @pl.when(jnp.logical_and(slot < SLOTS, g < NBLOCKS))
            def _():
                pltpu.async_copy(e_hbm.at[pl.ds(g * BLK, BLK)], ebuf, sem)
                pltpu.async_copy(
                    i_hbm.at[pl.ds(0, 1), pl.ds(g * BLK, BLK)], ibuf, sem
                )

        def finish(slot, ebuf, ibuf, sem):
            g = base_blk + slot

            @pl.when(g < NBLOCKS)
            def _():
                pltpu.make_async_copy(e_hbm.at[pl.ds(0, BLK)], ebuf, sem).wait()
                pltpu.make_async_copy(
                    i_hbm.at[pl.ds(0, 1), pl.ds(0, BLK)], ibuf, sem
                ).wait()
                pltpu.sync_copy(ebuf, acc.at[ibuf.at[0]], add=True)

        bufs = (
            (ebuf0, ibuf0, sem0),
            (ebuf1, ibuf1, sem1),
            (ebuf2, ibuf2, sem2),
        )
        for k in range(NBUF):
            start(k, *bufs[k])

        @pl.loop(0, SLOTS - 1, step=NBUF)
        def _(s0):
            for b in range(NBUF):
                slot = s0 + b
                finish(slot, *bufs[b])
                start(slot + NBUF, *bufs[b])

        finish(SLOTS - 1, *bufs[(SLOTS - 1) % NBUF])

        plsc.subcore_barrier()

        pltpu.sync_copy(
            acc.at[pl.ds(sid * RPT, RPT)], o_hbm.at[cid, pl.ds(sid * RPT, RPT)]
        )

    return sc_kernel(edge_attr, edge_index)


BR = 2000


def _tc_mlp(x, partials, W1a, W1b, W2, b1, b2):
    def body(x_ref, p_ref, w1a_ref, w1b_ref, w2_ref, b1_ref, b2_ref, o_ref):
        aggr = p_ref[0] + p_ref[1]
        h = (
            jnp.dot(x_ref[...], w1a_ref[...], preferred_element_type=jnp.float32)
            + jnp.dot(aggr, w1b_ref[...], preferred_element_type=jnp.float32)
            + b1_ref[...]
        )
        h = jnp.maximum(h, 0.0)
        o_ref[...] = (
            jnp.dot(h, w2_ref[...], preferred_element_type=jnp.float32) + b2_ref[...]
        )

    return pl.pallas_call(
        body,
        grid=(N_NODES // BR,),
        in_specs=[
            pl.BlockSpec((BR, D), lambda i: (i, 0)),
            pl.BlockSpec((NC, BR, D), lambda i: (0, i, 0)),
            pl.BlockSpec((D, D), lambda i: (0, 0)),
            pl.BlockSpec((D, D), lambda i: (0, 0)),
            pl.BlockSpec((D, D), lambda i: (0, 0)),
            pl.BlockSpec((1, D), lambda i: (0, 0)),
            pl.BlockSpec((1, D), lambda i: (0, 0)),
        ],
        out_specs=pl.BlockSpec((BR, D), lambda i: (i, 0)),
        out_shape=jax.ShapeDtypeStruct((N_NODES, D), jnp.float32),
    )(x, partials, W1a, W1b, W2, b1, b2)


def kernel(x, edge_index, edge_attr, u, batch, W1, b1, W2, b2):
    partials = _sc_segment_sum(edge_attr, edge_index.astype(jnp.int32))
    return _tc_mlp(
        x, partials, W1[:D], W1[D:], W2, b1.reshape(1, D), b2.reshape(1, D)
    )


# speedup vs baseline: 9.2756x; 1.0058x over previous
import functools

import jax
import jax.numpy as jnp
from jax import lax
from jax.experimental import pallas as pl
from jax.experimental.pallas import tpu as pltpu
from jax.experimental.pallas import tpu_sc as plsc

N_NODES = 10000
N_EDGES = 320000
D = 128

NC, NS = 2, 16
NW = NC * NS
BLK = 128
NBLOCKS = N_EDGES // BLK
SLOTS = 79
NBUF = 3
N_PAD = 10112
RPT = N_PAD // NS


def _sc_segment_sum(edge_attr, edge_index):
    mesh = plsc.VectorSubcoreMesh(core_axis_name="c", subcore_axis_name="s")

    @functools.partial(
        pl.kernel,
        out_type=jax.ShapeDtypeStruct((NC, N_PAD, D), jnp.float32),
        mesh=mesh,
        scratch_types=[
            pltpu.VMEM_SHARED((N_PAD, D), jnp.float32),
            pltpu.VMEM((BLK, D), jnp.float32),
            pltpu.VMEM((BLK, D), jnp.float32),
            pltpu.VMEM((BLK, D), jnp.float32),
            pltpu.VMEM((1, BLK), jnp.int32),
            pltpu.VMEM((1, BLK), jnp.int32),
            pltpu.VMEM((1, BLK), jnp.int32),
            pltpu.SemaphoreType.DMA,
            pltpu.SemaphoreType.DMA,
            pltpu.SemaphoreType.DMA,
        ],
    )
    def sc_kernel(
        e_hbm, i_hbm, o_hbm, acc,
        ebuf0, ebuf1, ebuf2,
        ibuf0, ibuf1, ibuf2,
        sem0, sem1, sem2,
    ):
        cid = lax.axis_index("c")
        sid = lax.axis_index("s")
        wid = sid * NC + cid
        base_blk = wid * SLOTS

        @pl.loop(0, BLK)
        def _(r):
            @pl.loop(0, D, step=16)
            def _(j):
                ebuf0[r, pl.ds(j, 16)] = jnp.zeros((16,), jnp.float32)

        @pl.loop(0, RPT - BLK + 1, step=BLK)
        def _(r0):
            pltpu.sync_copy(ebuf0, acc.at[pl.ds(sid * RPT + r0, BLK)])

        pltpu.sync_copy(
            ebuf0.at[pl.ds(0, RPT - 4 * BLK)],
            acc.at[pl.ds(sid * RPT + 4 * BLK, RPT - 4 * BLK)],
        )

        plsc.subcore_barrier()

        def start(slot, ebuf, ibuf, sem):
            g = base_blk + slot

            @pl.when(jnp.logical_and(slot < SLOTS, g < NBLOCKS))
            def _():
                pltpu.async_copy(e_hbm.at[pl.ds(g * BLK, BLK)], ebuf, sem)
                pltpu.async_copy(
                    i_hbm.at[pl.ds(0, 1), pl.ds(g * BLK, BLK)], ibuf, sem
                )

        def finish(slot, ebuf, ibuf, sem):
            g = base_blk + slot

            @pl.when(g < NBLOCKS)
            def _():
                pltpu.make_async_copy(e_hbm.at[pl.ds(0, BLK)], ebuf, sem).wait()
                pltpu.make_async_copy(
                    i_hbm.at[pl.ds(0, 1), pl.ds(0, BLK)], ibuf, sem
                ).wait()
                pltpu.sync_copy(ebuf, acc.at[ibuf.at[0]], add=True)

        bufs = (
            (ebuf0, ibuf0, sem0),
            (ebuf1, ibuf1, sem1),
            (ebuf2, ibuf2, sem2),
        )
        for k in range(NBUF):
            start(k, *bufs[k])

        @pl.loop(0, SLOTS - 1, step=NBUF)
        def _(s0):
            for b in range(NBUF):
                slot = s0 + b
                finish(slot, *bufs[b])
                start(slot + NBUF, *bufs[b])

        finish(SLOTS - 1, *bufs[(SLOTS - 1) % NBUF])

        plsc.subcore_barrier()

        pltpu.sync_copy(
            acc.at[pl.ds(sid * RPT, RPT)], o_hbm.at[cid, pl.ds(sid * RPT, RPT)]
        )

    return sc_kernel(edge_attr, edge_index)


BR = 5000


def _tc_mlp(x, partials, W1a, W1b, W2, b1, b2):
    def body(x_ref, p_ref, w1a_ref, w1b_ref, w2_ref, b1_ref, b2_ref, o_ref):
        aggr = p_ref[0] + p_ref[1]
        h = (
            jnp.dot(x_ref[...], w1a_ref[...], preferred_element_type=jnp.float32)
            + jnp.dot(aggr, w1b_ref[...], preferred_element_type=jnp.float32)
            + b1_ref[...]
        )
        h = jnp.maximum(h, 0.0)
        o_ref[...] = (
            jnp.dot(h, w2_ref[...], preferred_element_type=jnp.float32) + b2_ref[...]
        )

    return pl.pallas_call(
        body,
        grid=(N_NODES // BR,),
        in_specs=[
            pl.BlockSpec((BR, D), lambda i: (i, 0)),
            pl.BlockSpec((NC, BR, D), lambda i: (0, i, 0)),
            pl.BlockSpec((D, D), lambda i: (0, 0)),
            pl.BlockSpec((D, D), lambda i: (0, 0)),
            pl.BlockSpec((D, D), lambda i: (0, 0)),
            pl.BlockSpec((1, D), lambda i: (0, 0)),
            pl.BlockSpec((1, D), lambda i: (0, 0)),
        ],
        out_specs=pl.BlockSpec((BR, D), lambda i: (i, 0)),
        out_shape=jax.ShapeDtypeStruct((N_NODES, D), jnp.float32),
    )(x, partials, W1a, W1b, W2, b1, b2)


def kernel(x, edge_index, edge_attr, u, batch, W1, b1, W2, b2):
    partials = _sc_segment_sum(edge_attr, edge_index.astype(jnp.int32))
    return _tc_mlp(
        x, partials, W1[:D], W1[D:], W2, b1.reshape(1, D), b2.reshape(1, D)
    )


# speedup vs baseline: 9.4685x; 1.0208x over previous
import functools

import jax
import jax.numpy as jnp
from jax import lax
from jax.experimental import pallas as pl
from jax.experimental.pallas import tpu as pltpu
from jax.experimental.pallas import tpu_sc as plsc

N_NODES = 10000
N_EDGES = 320000
D = 128

NC, NS = 2, 16
NW = NC * NS
BLK = 128
NBLOCKS = N_EDGES // BLK
SLOTS = 79
NBUF = 3
N_PAD = 10112
RPT = N_PAD // NS


def _sc_segment_sum(edge_attr, edge_index):
    mesh = plsc.VectorSubcoreMesh(core_axis_name="c", subcore_axis_name="s")

    @functools.partial(
        pl.kernel,
        out_type=jax.ShapeDtypeStruct((NC, N_PAD, D), jnp.float32),
        mesh=mesh,
        scratch_types=[
            pltpu.VMEM_SHARED((N_PAD, D), jnp.float32),
            pltpu.VMEM((BLK, D), jnp.float32),
            pltpu.VMEM((BLK, D), jnp.float32),
            pltpu.VMEM((BLK, D), jnp.float32),
            pltpu.VMEM((1, BLK), jnp.int32),
            pltpu.VMEM((1, BLK), jnp.int32),
            pltpu.VMEM((1, BLK), jnp.int32),
            pltpu.SemaphoreType.DMA,
            pltpu.SemaphoreType.DMA,
            pltpu.SemaphoreType.DMA,
        ],
    )
    def sc_kernel(
        e_hbm, i_hbm, o_hbm, acc,
        ebuf0, ebuf1, ebuf2,
        ibuf0, ibuf1, ibuf2,
        sem0, sem1, sem2,
    ):
        cid = lax.axis_index("c")
        sid = lax.axis_index("s")
        wid = sid * NC + cid
        base_blk = wid * SLOTS

        bufs = (
            (ebuf1, ibuf1, sem1),
            (ebuf2, ibuf2, sem2),
            (ebuf0, ibuf0, sem0),
        )

        def start(slot, ebuf, ibuf, sem):
            g = base_blk + slot

            @pl.when(jnp.logical_and(slot < SLOTS, g < NBLOCKS))
            def _():
                pltpu.async_copy(e_hbm.at[pl.ds(g * BLK, BLK)], ebuf, sem)
                pltpu.async_copy(
                    i_hbm.at[pl.ds(0, 1), pl.ds(g * BLK, BLK)], ibuf, sem
                )

        def finish(slot, ebuf, ibuf, sem):
            g = base_blk + slot

            @pl.when(g < NBLOCKS)
            def _():
                pltpu.make_async_copy(e_hbm.at[pl.ds(0, BLK)], ebuf, sem).wait()
                pltpu.make_async_copy(
                    i_hbm.at[pl.ds(0, 1), pl.ds(0, BLK)], ibuf, sem
                ).wait()
                pltpu.sync_copy(ebuf, acc.at[ibuf.at[0]], add=True)

        start(0, *bufs[0])
        start(1, *bufs[1])

        @pl.loop(0, BLK)
        def _(r):
            @pl.loop(0, D, step=16)
            def _(j):
                ebuf0[r, pl.ds(j, 16)] = jnp.zeros((16,), jnp.float32)

        @pl.loop(0, RPT - BLK + 1, step=BLK)
        def _(r0):
            pltpu.sync_copy(ebuf0, acc.at[pl.ds(sid * RPT + r0, BLK)])

        pltpu.sync_copy(
            ebuf0.at[pl.ds(0, RPT - 4 * BLK)],
            acc.at[pl.ds(sid * RPT + 4 * BLK, RPT - 4 * BLK)],
        )

        plsc.subcore_barrier()

        start(2, *bufs[2])

        @pl.loop(0, SLOTS - 1, step=NBUF)
        def _(s0):
            for b in range(NBUF):
                slot = s0 + b
                finish(slot, *bufs[b])
                start(slot + NBUF, *bufs[b])

        finish(SLOTS - 1, *bufs[(SLOTS - 1) % NBUF])

        plsc.subcore_barrier()

        pltpu.sync_copy(
            acc.at[pl.ds(sid * RPT, RPT)], o_hbm.at[cid, pl.ds(sid * RPT, RPT)]
        )

    return sc_kernel(edge_attr, edge_index)


BR = 5000


def _tc_mlp(x, partials, W1a, W1b, W2, b1, b2):
    def body(x_ref, p_ref, w1a_ref, w1b_ref, w2_ref, b1_ref, b2_ref, o_ref):
        aggr = p_ref[0] + p_ref[1]
        h = (
            jnp.dot(x_ref[...], w1a_ref[...], preferred_element_type=jnp.float32)
            + jnp.dot(aggr, w1b_ref[...], preferred_element_type=jnp.float32)
            + b1_ref[...]
        )
        h = jnp.maximum(h, 0.0)
        o_ref[...] = (
            jnp.dot(h, w2_ref[...], preferred_element_type=jnp.float32) + b2_ref[...]
        )

    return pl.pallas_call(
        body,
        grid=(N_NODES // BR,),
        in_specs=[
            pl.BlockSpec((BR, D), lambda i: (i, 0)),
            pl.BlockSpec((NC, BR, D), lambda i: (0, i, 0)),
            pl.BlockSpec((D, D), lambda i: (0, 0)),
            pl.BlockSpec((D, D), lambda i: (0, 0)),
            pl.BlockSpec((D, D), lambda i: (0, 0)),
            pl.BlockSpec((1, D), lambda i: (0, 0)),
            pl.BlockSpec((1, D), lambda i: (0, 0)),
        ],
        out_specs=pl.BlockSpec((BR, D), lambda i: (i, 0)),
        out_shape=jax.ShapeDtypeStruct((N_NODES, D), jnp.float32),
    )(x, partials, W1a, W1b, W2, b1, b2)


def kernel(x, edge_index, edge_attr, u, batch, W1, b1, W2, b2):
    partials = _sc_segment_sum(edge_attr, edge_index.astype(jnp.int32))
    return _tc_mlp(
        x, partials, W1[:D], W1[D:], W2, b1.reshape(1, D), b2.reshape(1, D)
    )
